# baseline (speedup 1.0000x reference)
import functools

import jax
import jax.numpy as jnp
from jax import lax
from jax.experimental import pallas as pl
from jax.experimental.pallas import tpu as pltpu
from jax.experimental.pallas import tpu_sc as plsc

B = 1024
D = 128
V = 100000
K = 8
T = 12
VT = 256
NT = (V + VT - 1) // VT

_NEG_INF = float("-inf")
_BIG_I32 = 2**31 - 1


def _gather_rows(table, idx, n):
    info = plsc.get_sparse_core_info()
    nw = info.num_cores * info.num_subcores
    assert n % (8 * nw) == 0
    b_per_w = n // nw
    mesh = plsc.VectorSubcoreMesh(core_axis_name="c", subcore_axis_name="s")

    @functools.partial(
        pl.kernel,
        mesh=mesh,
        out_type=jax.ShapeDtypeStruct((n, D), jnp.float32),
        scratch_types=[
            pltpu.VMEM((b_per_w,), jnp.int32),
            pltpu.VMEM((b_per_w, D), jnp.float32),
            pltpu.SemaphoreType.DMA,
        ],
    )
    def k(table_hbm, idx_hbm, out_hbm, idx_v, rows_v, sem):
        wid = lax.axis_index("s") * info.num_cores + lax.axis_index("c")
        base = wid * b_per_w
        pltpu.sync_copy(idx_hbm.at[pl.ds(base, b_per_w)], idx_v)
        pltpu.async_copy(table_hbm.at[idx_v], rows_v, sem).wait()
        pltpu.sync_copy(rows_v, out_hbm.at[pl.ds(base, b_per_w)])

    return k(table, idx)


def _topk_body(emb_ref, w_ref, idx_out_ref, vals_scr, idx_scr, s_scr):
    j = pl.program_id(0)

    @pl.when(j == 0)
    def _():
        vals_scr[...] = jnp.full((B, T), _NEG_INF, jnp.float32)
        idx_scr[...] = jnp.full((B, T), _BIG_I32, jnp.int32)

    s = lax.dot_general(
        emb_ref[...], w_ref[...], (((1,), (1,)), ((), ())),
        preferred_element_type=jnp.float32,
    )
    col = jax.lax.broadcasted_iota(jnp.int32, (1, VT), 1) + j * VT
    s = jnp.where(col < V, s, _NEG_INF)
    s_scr[...] = s

    t0 = vals_scr[:, T - 1:T]
    c = jnp.sum((s > t0).astype(jnp.int32), axis=1, keepdims=True)
    n = jnp.minimum(jnp.max(c), T)
    col_t = jax.lax.broadcasted_iota(jnp.int32, (1, T), 1)

    def _insert(_, carry):
        sv = s_scr[...]
        pv = vals_scr[...]
        pi = idx_scr[...]
        m = jnp.max(sv, axis=1, keepdims=True)
        sel = jnp.min(jnp.where(sv == m, col, _BIG_I32), axis=1,
                      keepdims=True)
        s_scr[...] = jnp.where(col == sel, _NEG_INF, sv)
        do = m > pv[:, T - 1:T]
        p = jnp.sum((pv >= m).astype(jnp.int32), axis=1, keepdims=True)
        sh_v = jnp.concatenate([pv[:, :1], pv[:, :T - 1]], axis=1)
        sh_i = jnp.concatenate([pi[:, :1], pi[:, :T - 1]], axis=1)
        nv = jnp.where(col_t < p, pv, jnp.where(col_t == p, m, sh_v))
        ni = jnp.where(col_t < p, pi, jnp.where(col_t == p, sel, sh_i))
        vals_scr[...] = jnp.where(do, nv, pv)
        idx_scr[...] = jnp.where(do, ni, pi)
        return carry

    lax.fori_loop(0, n, _insert, 0)
    cand_v = vals_scr[...]
    cand_i = idx_scr[...]

    @pl.when(j == NT - 1)
    def _():
        u = jnp.exp(cand_v - cand_v[:, 0:1])
        ub = jax.lax.bitcast_convert_type(u, jnp.int32)
        rem = jnp.full((B, T), True, jnp.bool_)
        picks = []
        for _ in range(K):
            ubm = jnp.where(rem, ub, -_BIG_I32)
            mx = jnp.max(ubm, axis=1, keepdims=True)
            cls = rem & (ubm >= mx - 1)
            pick = jnp.min(jnp.where(cls, cand_i, _BIG_I32), axis=1,
                           keepdims=True)
            picks.append(pick)
            rem = rem & (cand_i != pick)
        idx_out_ref[...] = jnp.concatenate(picks, axis=1)


def _topk_pallas(emb, w, interpret=False):
    return pl.pallas_call(
        _topk_body,
        grid=(NT,),
        in_specs=[
            pl.BlockSpec((B, D), lambda j: (0, 0)),
            pl.BlockSpec((VT, D), lambda j: (j, 0)),
        ],
        out_specs=pl.BlockSpec((B, K), lambda j: (0, 0)),
        out_shape=jax.ShapeDtypeStruct((B, K), jnp.int32),
        scratch_shapes=[
            pltpu.VMEM((B, T), jnp.float32),
            pltpu.VMEM((B, T), jnp.int32),
            pltpu.VMEM((B, VT), jnp.float32),
        ],
        interpret=interpret,
    )(emb, w)


def _att_body(h_ref, a_ref, bt_ref, out_ref):
    h = h_ref[...]
    ha = jnp.tanh(
        lax.dot_general(h, a_ref[...], (((1,), (0,)), ((), ())),
                        preferred_element_type=jnp.float32,
                        precision=lax.Precision.HIGHEST))
    e = jnp.sum(ha * bt_ref[...], axis=1, keepdims=True)
    e3 = e.reshape(B, K, 1)
    p = jnp.exp(e3 - jnp.max(e3, axis=1, keepdims=True))
    wgt = p / jnp.sum(p, axis=1, keepdims=True)
    out_ref[...] = jnp.sum(h.reshape(B, K, D) * wgt, axis=1)


def _att_pallas(h, a, bt, interpret=False):
    return pl.pallas_call(
        _att_body,
        out_shape=jax.ShapeDtypeStruct((B, D), jnp.float32),
        interpret=interpret,
    )(h, a, bt)


def kernel(conceptnet_text_vec, W, a, b):
    emb = _gather_rows(W, conceptnet_text_vec, B)
    idx2 = _topk_pallas(emb, W)
    h = _gather_rows(W, idx2.reshape(-1), B * K)
    return _att_pallas(h, a, b.reshape(1, D))


# speedup vs baseline: 1.7027x; 1.7027x over previous
import functools

import jax
import jax.numpy as jnp
from jax import lax
from jax.experimental import pallas as pl
from jax.experimental.pallas import tpu as pltpu
from jax.experimental.pallas import tpu_sc as plsc

B = 1024
D = 128
V = 100000
K = 8
T = 12
VT = 1024
NT = (V + VT - 1) // VT

_NEG_INF = float("-inf")
_BIG_I32 = 2**31 - 1


def _gather_rows(table, idx, n):
    info = plsc.get_sparse_core_info()
    nw = info.num_cores * info.num_subcores
    assert n % (8 * nw) == 0
    b_per_w = n // nw
    mesh = plsc.VectorSubcoreMesh(core_axis_name="c", subcore_axis_name="s")

    @functools.partial(
        pl.kernel,
        mesh=mesh,
        out_type=jax.ShapeDtypeStruct((n, D), jnp.float32),
        scratch_types=[
            pltpu.VMEM((b_per_w,), jnp.int32),
            pltpu.VMEM((b_per_w, D), jnp.float32),
            pltpu.SemaphoreType.DMA,
        ],
    )
    def k(table_hbm, idx_hbm, out_hbm, idx_v, rows_v, sem):
        wid = lax.axis_index("s") * info.num_cores + lax.axis_index("c")
        base = wid * b_per_w
        pltpu.sync_copy(idx_hbm.at[pl.ds(base, b_per_w)], idx_v)
        pltpu.async_copy(table_hbm.at[idx_v], rows_v, sem).wait()
        pltpu.sync_copy(rows_v, out_hbm.at[pl.ds(base, b_per_w)])

    return k(table, idx)


def _topk_body(emb_ref, w_ref, idx_out_ref, vals_scr, idx_scr, s_scr):
    j = pl.program_id(0)

    @pl.when(j == 0)
    def _():
        vals_scr[...] = jnp.full((B, T), _NEG_INF, jnp.float32)
        idx_scr[...] = jnp.full((B, T), _BIG_I32, jnp.int32)

    s = lax.dot_general(
        emb_ref[...], w_ref[...], (((1,), (1,)), ((), ())),
        preferred_element_type=jnp.float32,
    )
    col = jax.lax.broadcasted_iota(jnp.int32, (1, VT), 1) + j * VT
    s = jnp.where(col < V, s, _NEG_INF)
    s_scr[...] = s

    t0 = vals_scr[:, T - 1:T]
    c = jnp.sum((s > t0).astype(jnp.int32), axis=1, keepdims=True)
    n = jnp.minimum(jnp.max(c), T)
    col_t = jax.lax.broadcasted_iota(jnp.int32, (1, T), 1)

    def _insert(_, carry):
        sv = s_scr[...]
        pv = vals_scr[...]
        pi = idx_scr[...]
        m = jnp.max(sv, axis=1, keepdims=True)
        sel = jnp.min(jnp.where(sv == m, col, _BIG_I32), axis=1,
                      keepdims=True)
        s_scr[...] = jnp.where(col == sel, _NEG_INF, sv)
        do = m > pv[:, T - 1:T]
        p = jnp.sum((pv >= m).astype(jnp.int32), axis=1, keepdims=True)
        sh_v = jnp.concatenate([pv[:, :1], pv[:, :T - 1]], axis=1)
        sh_i = jnp.concatenate([pi[:, :1], pi[:, :T - 1]], axis=1)
        nv = jnp.where(col_t < p, pv, jnp.where(col_t == p, m, sh_v))
        ni = jnp.where(col_t < p, pi, jnp.where(col_t == p, sel, sh_i))
        vals_scr[...] = jnp.where(do, nv, pv)
        idx_scr[...] = jnp.where(do, ni, pi)
        return carry

    lax.fori_loop(0, n, _insert, 0)
    cand_v = vals_scr[...]
    cand_i = idx_scr[...]

    @pl.when(j == NT - 1)
    def _():
        u = jnp.exp(cand_v - cand_v[:, 0:1])
        ub = jax.lax.bitcast_convert_type(u, jnp.int32)
        rem = jnp.full((B, T), True, jnp.bool_)
        picks = []
        for _ in range(K):
            ubm = jnp.where(rem, ub, -_BIG_I32)
            mx = jnp.max(ubm, axis=1, keepdims=True)
            cls = rem & (ubm >= mx - 1)
            pick = jnp.min(jnp.where(cls, cand_i, _BIG_I32), axis=1,
                           keepdims=True)
            picks.append(pick)
            rem = rem & (cand_i != pick)
        idx_out_ref[...] = jnp.concatenate(picks, axis=1)


def _topk_pallas(emb, w, interpret=False):
    return pl.pallas_call(
        _topk_body,
        grid=(NT,),
        in_specs=[
            pl.BlockSpec((B, D), lambda j: (0, 0)),
            pl.BlockSpec((VT, D), lambda j: (j, 0)),
        ],
        out_specs=pl.BlockSpec((B, K), lambda j: (0, 0)),
        out_shape=jax.ShapeDtypeStruct((B, K), jnp.int32),
        scratch_shapes=[
            pltpu.VMEM((B, T), jnp.float32),
            pltpu.VMEM((B, T), jnp.int32),
            pltpu.VMEM((B, VT), jnp.float32),
        ],
        interpret=interpret,
    )(emb, w)


def _att_body(h_ref, a_ref, bt_ref, out_ref):
    h = h_ref[...]
    ha = jnp.tanh(
        lax.dot_general(h, a_ref[...], (((1,), (0,)), ((), ())),
                        preferred_element_type=jnp.float32,
                        precision=lax.Precision.HIGHEST))
    e = jnp.sum(ha * bt_ref[...], axis=1, keepdims=True)
    e3 = e.reshape(B, K, 1)
    p = jnp.exp(e3 - jnp.max(e3, axis=1, keepdims=True))
    wgt = p / jnp.sum(p, axis=1, keepdims=True)
    out_ref[...] = jnp.sum(h.reshape(B, K, D) * wgt, axis=1)


def _att_pallas(h, a, bt, interpret=False):
    return pl.pallas_call(
        _att_body,
        out_shape=jax.ShapeDtypeStruct((B, D), jnp.float32),
        interpret=interpret,
    )(h, a, bt)


def kernel(conceptnet_text_vec, W, a, b):
    emb = _gather_rows(W, conceptnet_text_vec, B)
    idx2 = _topk_pallas(emb, W)
    h = _gather_rows(W, idx2.reshape(-1), B * K)
    return _att_pallas(h, a, b.reshape(1, D))
